# baseline (speedup 1.0000x reference)
import functools

import jax
import jax.numpy as jnp
from jax import lax
from jax.experimental import pallas as pl
from jax.experimental.pallas import tpu as pltpu
from jax.experimental.pallas import tpu_sc as plsc

E = 8
K = 2
D = 768
DFF = 768
OUT = 768
TOK = 2048

BT = 128
NT = 40
P = NT * BT
CH = 512

NW = 32
TPW = TOK // NW
SPW = P // NW
GSUB = 4
GROWS = SPW // GSUB


def _route_body(x_ref, wg_ref, bg_ref, pos1_ref, pos2_ref, w1c_ref, w2c_ref,
                te_ref):
    x = x_ref[...]
    logits = jnp.dot(x, wg_ref[...], preferred_element_type=jnp.float32)
    logits = logits + bg_ref[0]
    lane = lax.broadcasted_iota(jnp.int32, (TOK, E), 1)
    m1 = jnp.max(logits, axis=1, keepdims=True)
    i1 = jnp.min(jnp.where(logits == m1, lane, E), axis=1, keepdims=True)
    l2 = jnp.where(lane == i1, -jnp.inf, logits)
    m2 = jnp.max(l2, axis=1, keepdims=True)
    i2 = jnp.min(jnp.where(l2 == m2, lane, E), axis=1, keepdims=True)
    t = jnp.exp(m2 - m1)
    wa = 1.0 / (1.0 + t)
    wb = 1.0 - wa
    O1 = (lane == i1).astype(jnp.float32)
    O2 = (lane == i2).astype(jnp.float32)

    li = lax.broadcasted_iota(jnp.int32, (CH, CH), 0)
    lj = lax.broadcasted_iota(jnp.int32, (CH, CH), 1)
    LT = (li > lj).astype(jnp.float32)
    run = jnp.zeros((1, E), jnp.float32)
    ranks = []
    for blk in (O1, O2):
        for c in range(TOK // CH):
            oc = blk[c * CH:(c + 1) * CH, :]
            ranks.append(jnp.dot(LT, oc, preferred_element_type=jnp.float32)
                         + run)
            run = run + jnp.sum(oc, axis=0, keepdims=True)
    rank1 = jnp.concatenate(ranks[: TOK // CH], axis=0)
    rank2 = jnp.concatenate(ranks[TOK // CH:], axis=0)

    cnt = run
    pad_cnt = jnp.ceil(cnt / BT) * BT
    ei = lax.broadcasted_iota(jnp.int32, (E, E), 0)
    ej = lax.broadcasted_iota(jnp.int32, (E, E), 1)
    UT = (ei < ej).astype(jnp.float32)
    off = jnp.dot(pad_cnt, UT, preferred_element_type=jnp.float32)

    pos1_ref[...] = jnp.sum(O1 * (off + rank1), axis=1,
                            keepdims=True).astype(jnp.int32)
    pos2_ref[...] = jnp.sum(O2 * (off + rank2), axis=1,
                            keepdims=True).astype(jnp.int32)
    w1c_ref[...] = wa
    w2c_ref[...] = wb

    toff = (lax.broadcasted_iota(jnp.int32, (NT, E), 0) * BT).astype(
        jnp.float32)
    eidx = lax.broadcasted_iota(jnp.int32, (NT, E), 1)
    ind = (toff >= off) & (toff < off + pad_cnt)
    te_ref[...] = jnp.sum(jnp.where(ind, eidx + 1, 0), axis=1,
                          keepdims=True) - 1


def _route(x, Wg, bg):
    return pl.pallas_call(
        _route_body,
        grid=(1,),
        in_specs=[
            pl.BlockSpec((TOK, D), lambda i: (0, 0)),
            pl.BlockSpec((D, E), lambda i: (0, 0)),
            pl.BlockSpec((1, E), lambda i: (0, 0)),
        ],
        out_specs=[
            pl.BlockSpec((TOK, 1), lambda i: (0, 0)),
            pl.BlockSpec((TOK, 1), lambda i: (0, 0)),
            pl.BlockSpec((TOK, 1), lambda i: (0, 0)),
            pl.BlockSpec((TOK, 1), lambda i: (0, 0)),
            pl.BlockSpec((NT, 1), lambda i: (0, 0)),
        ],
        out_shape=[
            jax.ShapeDtypeStruct((TOK, 1), jnp.int32),
            jax.ShapeDtypeStruct((TOK, 1), jnp.int32),
            jax.ShapeDtypeStruct((TOK, 1), jnp.float32),
            jax.ShapeDtypeStruct((TOK, 1), jnp.float32),
            jax.ShapeDtypeStruct((NT, 1), jnp.int32),
        ],
    )(x, Wg, bg.reshape(1, E))


def _dispatch_kernel(x_hbm, pos1_hbm, pos2_hbm, w1c_hbm, w2c_hbm,
                     xd_hbm, wd_hbm,
                     idx1, idx2, wbuf, xbuf, sem):
    wid = lax.axis_index("s") * 2 + lax.axis_index("c")
    base = wid * TPW
    pltpu.sync_copy(pos1_hbm.at[pl.ds(base, TPW)], idx1)
    pltpu.sync_copy(pos2_hbm.at[pl.ds(base, TPW)], idx2)
    pltpu.async_copy(x_hbm.at[pl.ds(base, TPW)], xbuf, sem).wait()
    pltpu.sync_copy(xbuf, xd_hbm.at[idx1])
    pltpu.sync_copy(xbuf, xd_hbm.at[idx2])
    pltpu.sync_copy(w1c_hbm.at[pl.ds(base, TPW)], wbuf)
    pltpu.sync_copy(wbuf, wd_hbm.at[idx1])
    pltpu.sync_copy(w2c_hbm.at[pl.ds(base, TPW)], wbuf)
    pltpu.sync_copy(wbuf, wd_hbm.at[idx2])


def _dispatch(x, pos1, pos2, w1c, w2c):
    mesh = plsc.VectorSubcoreMesh(core_axis_name="c", subcore_axis_name="s")
    f = functools.partial(
        pl.kernel,
        out_type=[
            jax.ShapeDtypeStruct((P, D), jnp.float32),
            jax.ShapeDtypeStruct((P,), jnp.float32),
        ],
        mesh=mesh,
        scratch_types=[
            pltpu.VMEM((TPW,), jnp.int32),
            pltpu.VMEM((TPW,), jnp.int32),
            pltpu.VMEM((TPW,), jnp.float32),
            pltpu.VMEM((TPW, D), jnp.float32),
            pltpu.SemaphoreType.DMA,
        ],
    )(_dispatch_kernel)
    return f(x, pos1, pos2, w1c, w2c)


def _mm_body(te_ref, xd_ref, w1_ref, b1_ref, w2_ref, b2_ref, wd_ref, yd_ref):
    i = pl.program_id(0)
    te = te_ref[i]

    @pl.when(te >= 0)
    def _():
        xd = xd_ref[...]
        w1f = w1_ref[0, :D, :] + w1_ref[0, D:, :]
        h = jnp.maximum(
            jnp.dot(xd, w1f, preferred_element_type=jnp.float32)
            + b1_ref[0, 0], 0.0)
        y = jnp.dot(h, w2_ref[0], preferred_element_type=jnp.float32)
        yd_ref[...] = wd_ref[...] * (y + b2_ref[0, 0])


def _expert_mm(te, xd, wd, W1, b1, W2, b2):
    grid_spec = pltpu.PrefetchScalarGridSpec(
        num_scalar_prefetch=1,
        grid=(NT,),
        in_specs=[
            pl.BlockSpec((BT, D), lambda i, te: (i, 0)),
            pl.BlockSpec((1, 2 * D, DFF),
                         lambda i, te: (jnp.maximum(te[i], 0), 0, 0)),
            pl.BlockSpec((1, 1, DFF),
                         lambda i, te: (jnp.maximum(te[i], 0), 0, 0)),
            pl.BlockSpec((1, DFF, OUT),
                         lambda i, te: (jnp.maximum(te[i], 0), 0, 0)),
            pl.BlockSpec((1, 1, OUT),
                         lambda i, te: (jnp.maximum(te[i], 0), 0, 0)),
            pl.BlockSpec((BT, 1), lambda i, te: (i, 0)),
        ],
        out_specs=pl.BlockSpec((BT, OUT), lambda i, te: (i, 0)),
    )
    return pl.pallas_call(
        _mm_body,
        grid_spec=grid_spec,
        out_shape=jax.ShapeDtypeStruct((P, OUT), jnp.float32),
        compiler_params=pltpu.CompilerParams(
            dimension_semantics=("arbitrary",),
        ),
    )(te, xd, W1, b1.reshape(E, 1, DFF), W2, b2.reshape(E, 1, OUT), wd)


def _combine_kernel(yd_hbm, pos1_hbm, pos2_hbm, out_hbm,
                    idx1, idx2, buf1, buf2, sem):
    wid = lax.axis_index("s") * 2 + lax.axis_index("c")
    base = wid * TPW
    pltpu.sync_copy(pos1_hbm.at[pl.ds(base, TPW)], idx1)
    pltpu.sync_copy(pos2_hbm.at[pl.ds(base, TPW)], idx2)
    pltpu.async_copy(yd_hbm.at[idx1], buf1, sem).wait()
    pltpu.async_copy(yd_hbm.at[idx2], buf2, sem).wait()

    def _add(r, carry):
        for c in range(OUT // 16):
            buf1[r, pl.ds(c * 16, 16)] = (buf1[r, pl.ds(c * 16, 16)]
                                          + buf2[r, pl.ds(c * 16, 16)])
        return carry

    lax.fori_loop(0, TPW, _add, 0)
    pltpu.sync_copy(buf1, out_hbm.at[pl.ds(base, TPW)])


def _combine(yd, pos1, pos2):
    mesh = plsc.VectorSubcoreMesh(core_axis_name="c", subcore_axis_name="s")
    f = functools.partial(
        pl.kernel,
        out_type=jax.ShapeDtypeStruct((TOK, OUT), jnp.float32),
        mesh=mesh,
        scratch_types=[
            pltpu.VMEM((TPW,), jnp.int32),
            pltpu.VMEM((TPW,), jnp.int32),
            pltpu.VMEM((TPW, OUT), jnp.float32),
            pltpu.VMEM((TPW, OUT), jnp.float32),
            pltpu.SemaphoreType.DMA,
        ],
    )(_combine_kernel)
    return f(yd, pos1, pos2)


@jax.jit
def kernel(inputs, Wg, bg, W1, b1, W2, b2):
    pos1, pos2, w1c, w2c, te = _route(inputs, Wg, bg)
    pos1 = pos1.reshape(TOK)
    pos2 = pos2.reshape(TOK)
    xd, wd = _dispatch(inputs, pos1, pos2, w1c.reshape(TOK), w2c.reshape(TOK))
    yd = _expert_mm(te.reshape(NT), xd, wd.reshape(P, 1), W1, b1, W2, b2)
    return _combine(yd, pos1, pos2)


# speedup vs baseline: 1.5703x; 1.5703x over previous
import functools

import jax
import jax.numpy as jnp
from jax import lax
from jax.experimental import pallas as pl
from jax.experimental.pallas import tpu as pltpu
from jax.experimental.pallas import tpu_sc as plsc

E = 8
K = 2
D = 768
DFF = 768
OUT = 768
TOK = 2048

BT = 256
NT = 24
P = NT * BT
CH = 512

NW = 32
TPW = TOK // NW


def _route_body(x_ref, wg_ref, bg_ref, pos1_ref, pos2_ref, w1c_ref, w2c_ref,
                te_ref):
    x = x_ref[...]
    logits = jnp.dot(x, wg_ref[...], preferred_element_type=jnp.float32)
    logits = logits + bg_ref[0]
    lane = lax.broadcasted_iota(jnp.int32, (TOK, E), 1)
    m1 = jnp.max(logits, axis=1, keepdims=True)
    i1 = jnp.min(jnp.where(logits == m1, lane, E), axis=1, keepdims=True)
    l2 = jnp.where(lane == i1, -jnp.inf, logits)
    m2 = jnp.max(l2, axis=1, keepdims=True)
    i2 = jnp.min(jnp.where(l2 == m2, lane, E), axis=1, keepdims=True)
    t = jnp.exp(m2 - m1)
    wa = 1.0 / (1.0 + t)
    wb = 1.0 - wa
    O1 = (lane == i1).astype(jnp.float32)
    O2 = (lane == i2).astype(jnp.float32)

    li = lax.broadcasted_iota(jnp.int32, (CH, CH), 0)
    lj = lax.broadcasted_iota(jnp.int32, (CH, CH), 1)
    LT = (li > lj).astype(jnp.float32)
    run = jnp.zeros((1, E), jnp.float32)
    ranks = []
    for blk in (O1, O2):
        for c in range(TOK // CH):
            oc = blk[c * CH:(c + 1) * CH, :]
            ranks.append(jnp.dot(LT, oc, preferred_element_type=jnp.float32)
                         + run)
            run = run + jnp.sum(oc, axis=0, keepdims=True)
    rank1 = jnp.concatenate(ranks[: TOK // CH], axis=0)
    rank2 = jnp.concatenate(ranks[TOK // CH:], axis=0)

    cnt = run
    pad_cnt = jnp.ceil(cnt / BT) * BT
    ei = lax.broadcasted_iota(jnp.int32, (E, E), 0)
    ej = lax.broadcasted_iota(jnp.int32, (E, E), 1)
    UT = (ei < ej).astype(jnp.float32)
    off = jnp.dot(pad_cnt, UT, preferred_element_type=jnp.float32)

    pos1_ref[...] = jnp.sum(O1 * (off + rank1), axis=1,
                            keepdims=True).astype(jnp.int32)
    pos2_ref[...] = jnp.sum(O2 * (off + rank2), axis=1,
                            keepdims=True).astype(jnp.int32)
    w1c_ref[...] = wa
    w2c_ref[...] = wb

    toff = (lax.broadcasted_iota(jnp.int32, (NT, E), 0) * BT).astype(
        jnp.float32)
    eidx = lax.broadcasted_iota(jnp.int32, (NT, E), 1)
    ind = (toff >= off) & (toff < off + pad_cnt)
    te_ref[...] = jnp.sum(jnp.where(ind, eidx + 1, 0), axis=1,
                          keepdims=True) - 1


def _route(x, Wg, bg):
    return pl.pallas_call(
        _route_body,
        grid=(1,),
        in_specs=[
            pl.BlockSpec((TOK, D), lambda i: (0, 0)),
            pl.BlockSpec((D, E), lambda i: (0, 0)),
            pl.BlockSpec((1, E), lambda i: (0, 0)),
        ],
        out_specs=[
            pl.BlockSpec((TOK, 1), lambda i: (0, 0)),
            pl.BlockSpec((TOK, 1), lambda i: (0, 0)),
            pl.BlockSpec((TOK, 1), lambda i: (0, 0)),
            pl.BlockSpec((TOK, 1), lambda i: (0, 0)),
            pl.BlockSpec((NT, 1), lambda i: (0, 0)),
        ],
        out_shape=[
            jax.ShapeDtypeStruct((TOK, 1), jnp.int32),
            jax.ShapeDtypeStruct((TOK, 1), jnp.int32),
            jax.ShapeDtypeStruct((TOK, 1), jnp.float32),
            jax.ShapeDtypeStruct((TOK, 1), jnp.float32),
            jax.ShapeDtypeStruct((NT, 1), jnp.int32),
        ],
    )(x, Wg, bg.reshape(1, E))


def _dispatch_kernel(x_hbm, pos1_hbm, pos2_hbm, xd_hbm,
                     idx1, idx2, xbuf, sem1, sem2, semx):
    wid = lax.axis_index("s") * 2 + lax.axis_index("c")
    base = wid * TPW
    c1 = pltpu.async_copy(pos1_hbm.at[pl.ds(base, TPW)], idx1, sem1)
    c2 = pltpu.async_copy(pos2_hbm.at[pl.ds(base, TPW)], idx2, sem2)
    cx = pltpu.async_copy(x_hbm.at[pl.ds(base, TPW)], xbuf, semx)
    c1.wait()
    c2.wait()
    cx.wait()
    s1 = pltpu.async_copy(xbuf, xd_hbm.at[idx1], sem1)
    s2 = pltpu.async_copy(xbuf, xd_hbm.at[idx2], sem2)
    s1.wait()
    s2.wait()


def _dispatch(x, pos1, pos2):
    mesh = plsc.VectorSubcoreMesh(core_axis_name="c", subcore_axis_name="s")
    f = functools.partial(
        pl.kernel,
        out_type=jax.ShapeDtypeStruct((P, D), jnp.float32),
        mesh=mesh,
        scratch_types=[
            pltpu.VMEM((TPW,), jnp.int32),
            pltpu.VMEM((TPW,), jnp.int32),
            pltpu.VMEM((TPW, D), jnp.float32),
            pltpu.SemaphoreType.DMA,
            pltpu.SemaphoreType.DMA,
            pltpu.SemaphoreType.DMA,
        ],
    )(_dispatch_kernel)
    return f(x, pos1, pos2)


def _mm_body(te_ref, xd_ref, w1_ref, b1_ref, w2_ref, b2_ref, yd_ref):
    i = pl.program_id(0)
    te = te_ref[i]

    @pl.when(te >= 0)
    def _():
        xd = xd_ref[...]
        w1f = w1_ref[0, :D, :] + w1_ref[0, D:, :]
        h = jnp.maximum(
            jnp.dot(xd, w1f, preferred_element_type=jnp.float32)
            + b1_ref[0, 0], 0.0)
        yd_ref[...] = (jnp.dot(h, w2_ref[0], preferred_element_type=jnp.float32)
                       + b2_ref[0, 0])


def _expert_mm(te, xd, W1, b1, W2, b2):
    grid_spec = pltpu.PrefetchScalarGridSpec(
        num_scalar_prefetch=1,
        grid=(NT,),
        in_specs=[
            pl.BlockSpec((BT, D), lambda i, te: (i, 0)),
            pl.BlockSpec((1, 2 * D, DFF),
                         lambda i, te: (jnp.maximum(te[i], 0), 0, 0)),
            pl.BlockSpec((1, 1, DFF),
                         lambda i, te: (jnp.maximum(te[i], 0), 0, 0)),
            pl.BlockSpec((1, DFF, OUT),
                         lambda i, te: (jnp.maximum(te[i], 0), 0, 0)),
            pl.BlockSpec((1, 1, OUT),
                         lambda i, te: (jnp.maximum(te[i], 0), 0, 0)),
        ],
        out_specs=pl.BlockSpec((BT, OUT), lambda i, te: (i, 0)),
    )
    return pl.pallas_call(
        _mm_body,
        grid_spec=grid_spec,
        out_shape=jax.ShapeDtypeStruct((P, OUT), jnp.float32),
        compiler_params=pltpu.CompilerParams(
            dimension_semantics=("arbitrary",),
        ),
    )(te, xd, W1, b1.reshape(E, 1, DFF), W2, b2.reshape(E, 1, OUT))


def _combine_kernel(yd_hbm, pos1_hbm, pos2_hbm, w1c_hbm, w2c_hbm, out_hbm,
                    idx1, idx2, wb1, wb2, buf1, buf2, sem1, sem2):
    wid = lax.axis_index("s") * 2 + lax.axis_index("c")
    base = wid * TPW
    pltpu.sync_copy(pos1_hbm.at[pl.ds(base, TPW)], idx1)
    pltpu.sync_copy(pos2_hbm.at[pl.ds(base, TPW)], idx2)
    g1 = pltpu.async_copy(yd_hbm.at[idx1], buf1, sem1)
    g2 = pltpu.async_copy(yd_hbm.at[idx2], buf2, sem2)
    pltpu.sync_copy(w1c_hbm.at[pl.ds(base, TPW)], wb1.at[pl.ds(0, TPW)])
    pltpu.sync_copy(w2c_hbm.at[pl.ds(base, TPW)], wb2.at[pl.ds(0, TPW)])
    g1.wait()
    g2.wait()

    def _wsum(r, carry):
        w1 = wb1[pl.ds(r, 16)][0]
        w2 = wb2[pl.ds(r, 16)][0]
        for c in range(OUT // 16):
            buf1[r, pl.ds(c * 16, 16)] = (w1 * buf1[r, pl.ds(c * 16, 16)]
                                          + w2 * buf2[r, pl.ds(c * 16, 16)])
        return carry

    lax.fori_loop(0, TPW, _wsum, 0)
    pltpu.sync_copy(buf1, out_hbm.at[pl.ds(base, TPW)])


def _combine(yd, pos1, pos2, w1c, w2c):
    mesh = plsc.VectorSubcoreMesh(core_axis_name="c", subcore_axis_name="s")
    f = functools.partial(
        pl.kernel,
        out_type=jax.ShapeDtypeStruct((TOK, OUT), jnp.float32),
        mesh=mesh,
        scratch_types=[
            pltpu.VMEM((TPW,), jnp.int32),
            pltpu.VMEM((TPW,), jnp.int32),
            pltpu.VMEM((TPW + 16,), jnp.float32),
            pltpu.VMEM((TPW + 16,), jnp.float32),
            pltpu.VMEM((TPW, OUT), jnp.float32),
            pltpu.VMEM((TPW, OUT), jnp.float32),
            pltpu.SemaphoreType.DMA,
            pltpu.SemaphoreType.DMA,
        ],
    )(_combine_kernel)
    return f(yd, pos1, pos2, w1c, w2c)


@jax.jit
def kernel(inputs, Wg, bg, W1, b1, W2, b2):
    pos1, pos2, w1c, w2c, te = _route(inputs, Wg, bg)
    pos1 = pos1.reshape(TOK)
    pos2 = pos2.reshape(TOK)
    xd = _dispatch(inputs, pos1, pos2)
    yd = _expert_mm(te.reshape(NT), xd, W1, b1, W2, b2)
    return _combine(yd, pos1, pos2, w1c.reshape(TOK), w2c.reshape(TOK))


# speedup vs baseline: 2.3113x; 1.4719x over previous
import jax
import jax.numpy as jnp
from jax import lax
from jax.experimental import pallas as pl
from jax.experimental.pallas import tpu as pltpu

E = 8
K = 2
D = 768
DFF = 768
OUT = 768
TOK = 2048


def _moe_body(x_ref, wg_ref, bg_ref, w1_ref, b1_ref, w2_ref, b2_ref, out_ref,
              ha_ref, hb_ref, mw_ref):
    e = pl.program_id(0)

    @pl.when(e == 0)
    def _():
        x = x_ref[...]
        logits = jnp.dot(x, wg_ref[...], preferred_element_type=jnp.float32)
        logits = logits + bg_ref[0]
        lane = lax.broadcasted_iota(jnp.int32, (TOK, E), 1)
        m1 = jnp.max(logits, axis=1, keepdims=True)
        i1 = jnp.min(jnp.where(logits == m1, lane, E), axis=1, keepdims=True)
        l2 = jnp.where(lane == i1, -jnp.inf, logits)
        m2 = jnp.max(l2, axis=1, keepdims=True)
        i2 = jnp.min(jnp.where(l2 == m2, lane, E), axis=1, keepdims=True)
        t = jnp.exp(m2 - m1)
        wa = 1.0 / (1.0 + t)
        wb = 1.0 - wa
        mw_ref[...] = jnp.where(lane == i1, wa, 0.0) + jnp.where(
            lane == i2, wb, 0.0)

    @pl.when(e < E)
    def _():
        w1f = w1_ref[0, :D, :] + w1_ref[0, D:, :]
        h = jnp.maximum(
            jnp.dot(x_ref[...], w1f, preferred_element_type=jnp.float32)
            + b1_ref[0, 0], 0.0)

        @pl.when(e % 2 == 0)
        def _():
            ha_ref[...] = h

        @pl.when(e % 2 == 1)
        def _():
            hb_ref[...] = h

    @pl.when(e > 0)
    def _():
        ep = e - 1
        mw = jnp.sum(
            mw_ref[...]
            * (lax.broadcasted_iota(jnp.int32, (TOK, E), 1) == ep).astype(
                jnp.float32),
            axis=1, keepdims=True)

        def consume(h_ref):
            y = jnp.dot(h_ref[...], w2_ref[0],
                        preferred_element_type=jnp.float32)
            contrib = mw * (y + b2_ref[0, 0])

            @pl.when(ep == 0)
            def _():
                out_ref[...] = contrib

            @pl.when(ep > 0)
            def _():
                out_ref[...] += contrib

        @pl.when(ep % 2 == 0)
        def _():
            consume(ha_ref)

        @pl.when(ep % 2 == 1)
        def _():
            consume(hb_ref)


def kernel(inputs, Wg, bg, W1, b1, W2, b2):
    bg2 = bg.reshape(1, E)
    b1r = b1.reshape(E, 1, DFF)
    b2r = b2.reshape(E, 1, OUT)
    out = pl.pallas_call(
        _moe_body,
        grid=(E + 1,),
        in_specs=[
            pl.BlockSpec((TOK, D), lambda e: (0, 0)),
            pl.BlockSpec((D, E), lambda e: (0, 0)),
            pl.BlockSpec((1, E), lambda e: (0, 0)),
            pl.BlockSpec((1, 2 * D, DFF),
                         lambda e: (jnp.minimum(e, E - 1), 0, 0)),
            pl.BlockSpec((1, 1, DFF), lambda e: (jnp.minimum(e, E - 1), 0, 0)),
            pl.BlockSpec((1, DFF, OUT),
                         lambda e: (jnp.maximum(e - 1, 0), 0, 0)),
            pl.BlockSpec((1, 1, OUT), lambda e: (jnp.maximum(e - 1, 0), 0, 0)),
        ],
        out_specs=pl.BlockSpec((TOK, OUT), lambda e: (0, 0)),
        out_shape=jax.ShapeDtypeStruct((TOK, OUT), jnp.float32),
        scratch_shapes=[
            pltpu.VMEM((TOK, DFF), jnp.float32),
            pltpu.VMEM((TOK, DFF), jnp.float32),
            pltpu.VMEM((TOK, E), jnp.float32),
        ],
        compiler_params=pltpu.CompilerParams(
            dimension_semantics=("arbitrary",),
        ),
    )(inputs, Wg, bg2, W1, b1r, W2, b2r)
    return out


kernel = jax.jit(kernel)
